# baseline (speedup 1.0000x reference)
import functools

import jax
import jax.numpy as jnp
from jax import lax
from jax.experimental import pallas as pl
from jax.experimental.pallas import tpu as pltpu
from jax.experimental.pallas import tpu_sc as plsc

_NC = 2
_NS = 16
_NW = _NC * _NS

_LROWS = 250048
_B = 4096
_F = 26
_D = 32


def _relayout_body(tT_hbm, L_hbm, tile_v, lrow_v,
                   si0, si1, si2, si3, so0, so1):
    wid = lax.axis_index("s") * _NC + lax.axis_index("c")
    nw = jnp.where(wid < 4, 245, 244)
    si = (si0, si1, si2, si3)
    so = (so0, so1)
    iota = lax.iota(jnp.int32, 16)

    def in_desc(k, tb):
        lane_off = pl.multiple_of((wid + 32 * k) * 128, 128)
        return pltpu.make_async_copy(
            tT_hbm.at[:, pl.ds(lane_off, 128)], tile_v.at[tb], si[tb])

    def out_desc(k, lb):
        lrow_off = pl.multiple_of((wid + 32 * k) * 32, 32)
        return pltpu.make_async_copy(
            lrow_v.at[lb], L_hbm.at[pl.ds(lrow_off, 32)], so[lb])

    def transpose_col(tb, lb):
        def jj_body(jj):
            c0 = jnp.full((16,), 4 * jj, jnp.int32)
            for q in range(4):
                cvec = c0 + q
                for h in range(2):
                    rvec = iota + h * 16
                    v = plsc.load_gather(tile_v.at[tb], [rvec, cvec])
                    lrow_v[lb, jj, pl.ds(q * 32 + h * 16, 16)] = v

        plsc.parallel_loop(0, 32, unroll=16)(jj_body)

    def process(k, m):
        tb, lb = m, m & 1
        in_desc(k, tb).wait()

        @pl.when(k >= 2)
        def _wait_prev_out():
            out_desc(k - 2, lb).wait()

        transpose_col(tb, lb)
        out_desc(k, lb).start()

        @pl.when(k + 4 < nw)
        def _next_in():
            in_desc(k + 4, tb).start()

    for m in range(4):
        in_desc(m, m).start()

    def body(k):
        process(k, 0)
        for m in range(1, 4):
            @pl.when(k + m < nw)
            def _rest(k=k, m=m):
                process(k + m, m)

    pl.loop(0, nw, step=4)(body)
    out_desc(0, 0).wait()
    out_desc(0, 1).wait()


def _gather_body(xf_hbm, L_hbm, op_hbm, xblk_v, ridx_v, off_v, g_v, tbuf_v,
                 sg0, sg1, sg2, sg3, sv0, sv1, sv2, sv3):
    wid = lax.axis_index("s") * _NC + lax.axis_index("c")
    iota = lax.iota(jnp.int32, 16)
    sg = (sg0, sg1, sg2, sg3)
    sv = (sv0, sv1, sv2, sv3)
    pltpu.sync_copy(xf_hbm.at[wid], xblk_v)

    def fidx_body(f):
        for bl0 in range(0, 128, 16):
            p = iota * 26 + (bl0 * 26 + f)
            v = plsc.load_gather(xblk_v, [p >> 7, p & 127])
            ridx_v[f, pl.ds(bl0, 16)] = v >> 2
            off_v[f, pl.ds(bl0, 16)] = (v & 3) * 32

    plsc.parallel_loop(0, _F, unroll=2)(fidx_body)

    lane_off = pl.multiple_of(wid * 128, 128)

    def g_desc(f, gb):
        return pltpu.make_async_copy(L_hbm.at[ridx_v.at[f]], g_v.at[gb], sg[gb])

    def o_desc(f, ob):
        return pltpu.make_async_copy(
            tbuf_v.at[ob], op_hbm.at[f, :, pl.ds(lane_off, 128)], sv[ob])

    def process(f, m):
        gb = ob = m
        g_desc(f, m).wait()

        @pl.when(f + 3 < _F)
        def _next_g():
            g_desc(f + 3, (m + 3) & 3).start()

        @pl.when(f >= 4)
        def _wait_prev_out():
            o_desc(f - 4, ob).wait()

        def bl_body(bl0):
            offv = off_v[f, pl.ds(bl0, 16)]
            rowv = iota + bl0
            for d in range(_D):
                v = plsc.load_gather(g_v.at[gb], [rowv, offv + d])
                tbuf_v[ob, d, pl.ds(bl0, 16)] = v

        plsc.parallel_loop(0, 128, step=16, unroll=8)(bl_body)
        o_desc(f, ob).start()

    for m in range(3):
        g_desc(m, m).start()

    def body(f):
        process(f, 0)
        process(f + 1, 1)

        @pl.when(f + 2 < _F)
        def _m2():
            process(f + 2, 2)

        @pl.when(f + 3 < _F)
        def _m3():
            process(f + 3, 3)

    pl.loop(0, _F, step=4)(body)
    for m in range(4):
        o_desc(0, m).wait()


@jax.jit
def kernel(x, table):
    mesh = plsc.VectorSubcoreMesh(core_axis_name="c", subcore_axis_name="s")
    params = pltpu.CompilerParams(use_tc_tiling_on_sc=True, needs_layout_passes=False)

    tT = table.T
    L = pl.kernel(
        _relayout_body,
        out_type=jax.ShapeDtypeStruct((_LROWS, 128), jnp.float32),
        mesh=mesh,
        scratch_types=[
            pltpu.VMEM((4, 32, 128), jnp.float32),
            pltpu.VMEM((2, 32, 128), jnp.float32),
            pltpu.SemaphoreType.DMA,
            pltpu.SemaphoreType.DMA,
            pltpu.SemaphoreType.DMA,
            pltpu.SemaphoreType.DMA,
            pltpu.SemaphoreType.DMA,
            pltpu.SemaphoreType.DMA,
        ],
        compiler_params=params,
    )(tT)
    tailL = table[999936:1000000].reshape(16, 128)
    L = lax.dynamic_update_slice(L, tailL, (249984, 0))

    xf3 = x.reshape(_NW, _F, 128)
    op = pl.kernel(
        _gather_body,
        out_type=jax.ShapeDtypeStruct((_F, _D, _B), jnp.float32),
        mesh=mesh,
        scratch_types=[
            pltpu.VMEM((_F, 128), jnp.int32),
            pltpu.VMEM((_F, 128), jnp.int32),
            pltpu.VMEM((_F, 128), jnp.int32),
            pltpu.VMEM((4, 128, 128), jnp.float32),
            pltpu.VMEM((4, _D, 128), jnp.float32),
            pltpu.SemaphoreType.DMA,
            pltpu.SemaphoreType.DMA,
            pltpu.SemaphoreType.DMA,
            pltpu.SemaphoreType.DMA,
            pltpu.SemaphoreType.DMA,
            pltpu.SemaphoreType.DMA,
            pltpu.SemaphoreType.DMA,
            pltpu.SemaphoreType.DMA,
        ],
        compiler_params=params,
    )(xf3, L)
    return jnp.transpose(op, (2, 0, 1))


# speedup vs baseline: 1.0234x; 1.0234x over previous
import functools

import jax
import jax.numpy as jnp
from jax import lax
from jax.experimental import pallas as pl
from jax.experimental.pallas import tpu as pltpu
from jax.experimental.pallas import tpu_sc as plsc

_NC = 2
_NS = 16
_NW = _NC * _NS

_LROWS = 250048
_B = 4096
_F = 26
_D = 32


def _relayout_body(tT_hbm, L_hbm, tile_v, lrow_v,
                   si0, si1, si2, si3, so0, so1):
    wid = lax.axis_index("s") * _NC + lax.axis_index("c")
    nw = jnp.where(wid < 4, 245, 244)
    si = (si0, si1, si2, si3)
    so = (so0, so1)
    iota = lax.iota(jnp.int32, 16)

    def in_desc(k, tb):
        lane_off = pl.multiple_of((wid + 32 * k) * 128, 128)
        return pltpu.make_async_copy(
            tT_hbm.at[:, pl.ds(lane_off, 128)], tile_v.at[tb], si[tb])

    def out_desc(k, lb):
        lrow_off = pl.multiple_of((wid + 32 * k) * 32, 32)
        return pltpu.make_async_copy(
            lrow_v.at[lb], L_hbm.at[pl.ds(lrow_off, 32)], so[lb])

    def transpose_col(tb, lb):
        def jj_body(jj):
            c0 = jnp.full((16,), 4 * jj, jnp.int32)
            for q in range(4):
                cvec = c0 + q
                for h in range(2):
                    rvec = iota + h * 16
                    v = plsc.load_gather(tile_v.at[tb], [rvec, cvec])
                    lrow_v[lb, jj, pl.ds(q * 32 + h * 16, 16)] = v

        plsc.parallel_loop(0, 32, unroll=8)(jj_body)

    def process(k, m):
        tb, lb = m, m & 1
        in_desc(k, tb).wait()

        @pl.when(k >= 2)
        def _wait_prev_out():
            out_desc(k - 2, lb).wait()

        transpose_col(tb, lb)
        out_desc(k, lb).start()

        @pl.when(k + 4 < nw)
        def _next_in():
            in_desc(k + 4, tb).start()

    for m in range(4):
        in_desc(m, m).start()

    def body(k):
        process(k, 0)
        for m in range(1, 4):
            @pl.when(k + m < nw)
            def _rest(k=k, m=m):
                process(k + m, m)

    pl.loop(0, nw, step=4)(body)
    out_desc(0, 0).wait()
    out_desc(0, 1).wait()


def _gather_body(xf_hbm, L_hbm, op_hbm, xblk_v, ridx_v, off_v, g_v, tbuf_v,
                 sg0, sg1, sg2, sg3, sv0, sv1, sv2, sv3):
    wid = lax.axis_index("s") * _NC + lax.axis_index("c")
    iota = lax.iota(jnp.int32, 16)
    sg = (sg0, sg1, sg2, sg3)
    sv = (sv0, sv1, sv2, sv3)
    pltpu.sync_copy(xf_hbm.at[wid], xblk_v)

    def fidx_body(f):
        for bl0 in range(0, 128, 16):
            p = iota * 26 + (bl0 * 26 + f)
            v = plsc.load_gather(xblk_v, [p >> 7, p & 127])
            ridx_v[f, pl.ds(bl0, 16)] = v >> 2
            off_v[f, pl.ds(bl0, 16)] = (v & 3) * 32

    plsc.parallel_loop(0, _F, unroll=2)(fidx_body)

    lane_off = pl.multiple_of(wid * 128, 128)

    def g_desc(f, gb):
        return pltpu.make_async_copy(L_hbm.at[ridx_v.at[f]], g_v.at[gb], sg[gb])

    def o_desc(f, ob):
        return pltpu.make_async_copy(
            tbuf_v.at[ob], op_hbm.at[f, :, pl.ds(lane_off, 128)], sv[ob])

    def process(f, m):
        gb = ob = m
        g_desc(f, m).wait()

        @pl.when(f + 3 < _F)
        def _next_g():
            g_desc(f + 3, (m + 3) & 3).start()

        @pl.when(f >= 4)
        def _wait_prev_out():
            o_desc(f - 4, ob).wait()

        def bl_body(bl0):
            offv = off_v[f, pl.ds(bl0, 16)]
            rowv = iota + bl0
            for d in range(_D):
                v = plsc.load_gather(g_v.at[gb], [rowv, offv + d])
                tbuf_v[ob, d, pl.ds(bl0, 16)] = v

        plsc.parallel_loop(0, 128, step=16, unroll=4)(bl_body)
        o_desc(f, ob).start()

    for m in range(3):
        g_desc(m, m).start()

    def body(f):
        process(f, 0)
        process(f + 1, 1)

        @pl.when(f + 2 < _F)
        def _m2():
            process(f + 2, 2)

        @pl.when(f + 3 < _F)
        def _m3():
            process(f + 3, 3)

    pl.loop(0, _F, step=4)(body)
    for m in range(4):
        o_desc(0, m).wait()


@jax.jit
def kernel(x, table):
    mesh = plsc.VectorSubcoreMesh(core_axis_name="c", subcore_axis_name="s")
    params = pltpu.CompilerParams(use_tc_tiling_on_sc=True, needs_layout_passes=False)

    tT = table.T
    L = pl.kernel(
        _relayout_body,
        out_type=jax.ShapeDtypeStruct((_LROWS, 128), jnp.float32),
        mesh=mesh,
        scratch_types=[
            pltpu.VMEM((4, 32, 128), jnp.float32),
            pltpu.VMEM((2, 32, 128), jnp.float32),
            pltpu.SemaphoreType.DMA,
            pltpu.SemaphoreType.DMA,
            pltpu.SemaphoreType.DMA,
            pltpu.SemaphoreType.DMA,
            pltpu.SemaphoreType.DMA,
            pltpu.SemaphoreType.DMA,
        ],
        compiler_params=params,
    )(tT)
    tailL = table[999936:1000000].reshape(16, 128)
    L = lax.dynamic_update_slice(L, tailL, (249984, 0))

    xf3 = x.reshape(_NW, _F, 128)
    op = pl.kernel(
        _gather_body,
        out_type=jax.ShapeDtypeStruct((_F, _D, _B), jnp.float32),
        mesh=mesh,
        scratch_types=[
            pltpu.VMEM((_F, 128), jnp.int32),
            pltpu.VMEM((_F, 128), jnp.int32),
            pltpu.VMEM((_F, 128), jnp.int32),
            pltpu.VMEM((4, 128, 128), jnp.float32),
            pltpu.VMEM((4, _D, 128), jnp.float32),
            pltpu.SemaphoreType.DMA,
            pltpu.SemaphoreType.DMA,
            pltpu.SemaphoreType.DMA,
            pltpu.SemaphoreType.DMA,
            pltpu.SemaphoreType.DMA,
            pltpu.SemaphoreType.DMA,
            pltpu.SemaphoreType.DMA,
            pltpu.SemaphoreType.DMA,
        ],
        compiler_params=params,
    )(xf3, L)
    return jnp.transpose(op, (2, 0, 1))
